# baseline (speedup 1.0000x reference)
import functools

import jax
import jax.numpy as jnp
from jax import lax
from jax.experimental import pallas as pl
from jax.experimental.pallas import tpu as pltpu
from jax.experimental.pallas import tpu_sc as plsc

N_NODES = 10000
D_IN = 128
D_HID = 128
D_OUT = 64

NC, NS = 2, 16
NW = NC * NS
EC = 2048
COLS = D_HID // NW
NROW = 10240

TC_BLK = 2048
TC_GRID = NROW // TC_BLK


def _sc_mesh():
    return plsc.VectorSubcoreMesh(
        core_axis_name="c", subcore_axis_name="s", num_cores=NC, num_subcores=NS
    )


def _sc_degree(dst_f, nch):
    @functools.partial(
        pl.kernel,
        out_type=jax.ShapeDtypeStruct((NC, NS, NROW // 128, 128), jnp.float32),
        mesh=_sc_mesh(),
        compiler_params=pltpu.CompilerParams(needs_layout_passes=False),
        scratch_types=[
            pltpu.VMEM((NROW // 128, 128), jnp.float32),
            pltpu.VMEM((16, 128), jnp.int32),
        ],
    )
    def k(dst_hbm, out_hbm, deg_loc, dbuf):
        cid = lax.axis_index("c")
        sid = lax.axis_index("s")
        tid = cid * NS + sid
        nper = nch // NW

        def z(i, c):
            for cc in range(8):
                deg_loc[i, pl.ds(cc * 16, 16)] = jnp.zeros((16,), jnp.float32)
            return c

        lax.fori_loop(0, NROW // 128, z, 0)
        ones = jnp.ones((16,), jnp.float32)

        def chunk(q, c):
            pltpu.sync_copy(dst_hbm.at[tid * nper + q], dbuf)

            @plsc.parallel_loop(0, EC // 16, step=1, unroll=8)
            def inner(i):
                d16 = lax.bitwise_and(dbuf[i // 8, pl.ds((i % 8) * 16, 16)], 0xFFFF)
                plsc.addupdate_scatter(
                    deg_loc,
                    [lax.shift_right_logical(d16, 7), lax.bitwise_and(d16, 127)],
                    ones,
                )

            return c

        lax.fori_loop(0, nper, chunk, 0)
        pltpu.sync_copy(deg_loc, out_hbm.at[cid, sid])

    return k(dst_f)


def _sc_aggregate(yT_r, sd_f, nch):
    @functools.partial(
        pl.kernel,
        out_type=jax.ShapeDtypeStruct((NC, NS, COLS * NROW), jnp.float32),
        mesh=_sc_mesh(),
        compiler_params=pltpu.CompilerParams(needs_layout_passes=False),
        scratch_types=[
            pltpu.VMEM(((COLS // 2) * NROW,), jnp.int32),
            pltpu.VMEM((COLS * NROW,), jnp.float32),
            pltpu.VMEM((16, 128), jnp.int32),
            pltpu.VMEM((16, 128), jnp.int32),
            pltpu.SemaphoreType.DMA,
            pltpu.SemaphoreType.DMA,
        ],
    )
    def k(y_hbm, sd_hbm, out_hbm, y_loc, acc_loc, eA, eB, semA, semB):
        cid = lax.axis_index("c")
        sid = lax.axis_index("s")
        tid = cid * NS + sid

        pltpu.sync_copy(y_hbm.at[tid], y_loc)

        def z(i, carry):
            acc_loc[pl.ds(i * 16, 16)] = jnp.zeros((16,), jnp.float32)
            return carry

        lax.fori_loop(0, COLS * NROW // 16, z, 0)

        pltpu.async_copy(sd_hbm.at[0], eA, semA)

        def run_chunk(q, eb, sm, eo, smo):
            @pl.when(q + 1 < nch)
            def _pf():
                pltpu.async_copy(sd_hbm.at[q + 1], eo, smo)

            pltpu.make_async_copy(sd_hbm.at[q], eb, sm).wait()

            @plsc.parallel_loop(0, 16, step=1, unroll=2)
            def inner(i):
                for u in range(8):
                    sd16 = eb[i, pl.ds(u * 16, 16)]
                    s16 = lax.shift_right_logical(sd16, 16)
                    d16 = lax.bitwise_and(sd16, 0xFFFF)
                    for p in range(COLS // 2):
                        w = plsc.load_gather(y_loc, [s16 + (p * NROW)])
                        vlo = plsc.bitcast(lax.shift_left(w, 16), jnp.float32)
                        vhi = plsc.bitcast(
                            lax.bitwise_and(w, jnp.int32(-65536)), jnp.float32
                        )
                        plsc.addupdate_scatter(
                            acc_loc, [d16 + (2 * p * NROW)], vlo
                        )
                        plsc.addupdate_scatter(
                            acc_loc, [d16 + ((2 * p + 1) * NROW)], vhi
                        )

        def pairq(p, carry):
            run_chunk(2 * p, eA, semA, eB, semB)
            run_chunk(2 * p + 1, eB, semB, eA, semA)
            return carry

        lax.fori_loop(0, nch // 2, pairq, 0)

        pltpu.sync_copy(acc_loc, out_hbm.at[cid, sid])

    return k(yT_r, sd_f)


def _deg_inv(degp_ref):
    dsum = jnp.sum(degp_ref[...], axis=(0, 1)) + 1.0
    return lax.rsqrt(dsum)


def _tc_y_body(degp_ref, x_ref, wg_ref, y_ref):
    dinv = _deg_inv(degp_ref)
    xwT = lax.dot_general(
        wg_ref[...], x_ref[...], (((0,), (1,)), ((), ())),
        preferred_element_type=jnp.float32,
    )
    y_ref[...] = xwT * dinv


def _tc_head_body(degp_ref, agg_ref, y_ref, bg_ref, wl_ref, bl_ref, out_ref):
    dinv = _deg_inv(degp_ref)
    h = jnp.maximum((agg_ref[...] + y_ref[...]) * dinv + bg_ref[...], 0.0)
    out_ref[...] = (
        lax.dot_general(h, wl_ref[...], (((0,), (0,)), ((), ())),
                        preferred_element_type=jnp.float32)
        + bl_ref[...]
    )


def kernel(x, edge_index, W_g, b_g, W_l, b_l):
    src = edge_index[0].astype(jnp.int32)
    dst = edge_index[1].astype(jnp.int32)
    x_p = jnp.concatenate(
        [x, jnp.zeros((NROW - N_NODES, D_IN), jnp.float32)], axis=0
    )
    n_edges = src.shape[0]
    nch = -(-n_edges // EC)
    nch = -(-nch // NW) * NW
    pad = nch * EC - n_edges

    garbage = N_NODES + (jnp.arange(pad, dtype=jnp.int32) % 16)
    src_p = jnp.concatenate([src, jnp.zeros((pad,), jnp.int32)])
    dst_p = jnp.concatenate([dst, garbage])
    sd_f = (lax.shift_left(src_p, 16) | dst_p).reshape(nch, 16, 128)

    degp = _sc_degree(sd_f, nch).reshape(NC, NS, 1, NROW)

    yT = pl.pallas_call(
        _tc_y_body,
        grid=(TC_GRID,),
        in_specs=[
            pl.BlockSpec((NC, NS, 1, TC_BLK), lambda i: (0, 0, 0, i)),
            pl.BlockSpec((TC_BLK, D_IN), lambda i: (i, 0)),
            pl.BlockSpec((D_IN, D_HID), lambda i: (0, 0)),
        ],
        out_specs=pl.BlockSpec((D_HID, TC_BLK), lambda i: (0, i)),
        out_shape=jax.ShapeDtypeStruct((D_HID, NROW), jnp.float32),
    )(degp, x_p, W_g)

    yb = yT.astype(jnp.bfloat16)
    ypk = lax.bitcast_convert_type(
        jnp.stack([yb[0::2], yb[1::2]], axis=-1), jnp.int32
    )
    p = _sc_aggregate(ypk.reshape(NW, (COLS // 2) * NROW), sd_f, nch)
    aggT = p.reshape(D_HID, NROW)

    out = pl.pallas_call(
        _tc_head_body,
        grid=(TC_GRID,),
        in_specs=[
            pl.BlockSpec((NC, NS, 1, TC_BLK), lambda i: (0, 0, 0, i)),
            pl.BlockSpec((D_HID, TC_BLK), lambda i: (0, i)),
            pl.BlockSpec((D_HID, TC_BLK), lambda i: (0, i)),
            pl.BlockSpec((D_HID, 1), lambda i: (0, 0)),
            pl.BlockSpec((D_HID, D_OUT), lambda i: (0, 0)),
            pl.BlockSpec((1, D_OUT), lambda i: (0, 0)),
        ],
        out_specs=pl.BlockSpec((TC_BLK, D_OUT), lambda i: (i, 0)),
        out_shape=jax.ShapeDtypeStruct((NROW, D_OUT), jnp.float32),
    )(degp, aggT, yT, b_g.reshape(D_HID, 1), W_l, b_l.reshape(1, D_OUT))

    return out[:N_NODES]


# speedup vs baseline: 1.2693x; 1.2693x over previous
import functools

import jax
import jax.numpy as jnp
from jax import lax
from jax.experimental import pallas as pl
from jax.experimental.pallas import tpu as pltpu
from jax.experimental.pallas import tpu_sc as plsc

N_NODES = 10000
D_IN = 128
D_HID = 128
D_OUT = 64

NC, NS = 2, 16
NW = NC * NS
EC = 2048
COLS = D_HID // NW
NROW = 10240

TC_BLK = 2048
TC_GRID = NROW // TC_BLK


def _sc_mesh():
    return plsc.VectorSubcoreMesh(
        core_axis_name="c", subcore_axis_name="s", num_cores=NC, num_subcores=NS
    )


def _sc_degree(dst_f, nch):
    @functools.partial(
        pl.kernel,
        out_type=jax.ShapeDtypeStruct((NC, NS, NROW // 128, 128), jnp.float32),
        mesh=_sc_mesh(),
        compiler_params=pltpu.CompilerParams(needs_layout_passes=False),
        scratch_types=[
            pltpu.VMEM((NROW // 128, 128), jnp.float32),
            pltpu.VMEM((16, 128), jnp.int32),
        ],
    )
    def k(dst_hbm, out_hbm, deg_loc, dbuf):
        cid = lax.axis_index("c")
        sid = lax.axis_index("s")
        tid = cid * NS + sid
        nper = nch // NW

        def z(i, c):
            for cc in range(8):
                deg_loc[i, pl.ds(cc * 16, 16)] = jnp.zeros((16,), jnp.float32)
            return c

        lax.fori_loop(0, NROW // 128, z, 0)
        ones = jnp.ones((16,), jnp.float32)

        def chunk(q, c):
            pltpu.sync_copy(dst_hbm.at[tid * nper + q], dbuf)

            @plsc.parallel_loop(0, EC // 16, step=1, unroll=8)
            def inner(i):
                d16 = lax.bitwise_and(dbuf[i // 8, pl.ds((i % 8) * 16, 16)], 0xFFFF)
                plsc.addupdate_scatter(
                    deg_loc,
                    [lax.shift_right_logical(d16, 7), lax.bitwise_and(d16, 127)],
                    ones,
                )

            return c

        lax.fori_loop(0, nper, chunk, 0)
        pltpu.sync_copy(deg_loc, out_hbm.at[cid, sid])

    return k(dst_f)


def _sc_aggregate(yT_r, sd_f, nch):
    @functools.partial(
        pl.kernel,
        out_type=jax.ShapeDtypeStruct((NC, NS, COLS * NROW), jnp.float32),
        mesh=_sc_mesh(),
        compiler_params=pltpu.CompilerParams(needs_layout_passes=False),
        scratch_types=[
            pltpu.VMEM((COLS * NROW,), jnp.float32),
            pltpu.VMEM((COLS * NROW,), jnp.float32),
            pltpu.VMEM((16, 128), jnp.int32),
            pltpu.VMEM((16, 128), jnp.int32),
            pltpu.SemaphoreType.DMA,
            pltpu.SemaphoreType.DMA,
        ],
    )
    def k(y_hbm, sd_hbm, out_hbm, y_loc, acc_loc, eA, eB, semA, semB):
        cid = lax.axis_index("c")
        sid = lax.axis_index("s")
        tid = cid * NS + sid

        pltpu.sync_copy(y_hbm.at[tid], y_loc)

        def z(i, carry):
            acc_loc[pl.ds(i * 16, 16)] = jnp.zeros((16,), jnp.float32)
            return carry

        lax.fori_loop(0, COLS * NROW // 16, z, 0)

        pltpu.async_copy(sd_hbm.at[0], eA, semA)

        def run_chunk(q, eb, sm, eo, smo):
            @pl.when(q + 1 < nch)
            def _pf():
                pltpu.async_copy(sd_hbm.at[q + 1], eo, smo)

            pltpu.make_async_copy(sd_hbm.at[q], eb, sm).wait()

            @plsc.parallel_loop(0, EC // 16, step=1, unroll=8)
            def inner(i):
                sd16 = eb[i // 8, pl.ds((i % 8) * 16, 16)]
                s16 = lax.shift_right_logical(sd16, 16)
                d16 = lax.bitwise_and(sd16, 0xFFFF)
                for c in range(COLS):
                    v = plsc.load_gather(y_loc, [s16 + (c * NROW)])
                    plsc.addupdate_scatter(acc_loc, [d16 + (c * NROW)], v)

        def pairq(p, carry):
            run_chunk(2 * p, eA, semA, eB, semB)
            run_chunk(2 * p + 1, eB, semB, eA, semA)
            return carry

        lax.fori_loop(0, nch // 2, pairq, 0)

        pltpu.sync_copy(acc_loc, out_hbm.at[cid, sid])

    return k(yT_r, sd_f)


def _deg_inv(degp_ref):
    dsum = jnp.sum(degp_ref[...], axis=(0, 1)) + 1.0
    return lax.rsqrt(dsum)


def _tc_y_body(degp_ref, x_ref, wg_ref, y_ref):
    dinv = _deg_inv(degp_ref)
    xwT = lax.dot_general(
        wg_ref[...], x_ref[...], (((0,), (1,)), ((), ())),
        preferred_element_type=jnp.float32,
    )
    y_ref[...] = xwT * dinv


def _tc_head_body(degp_ref, agg_ref, y_ref, bg_ref, wl_ref, bl_ref, out_ref):
    dinv = _deg_inv(degp_ref)
    h = jnp.maximum((agg_ref[...] + y_ref[...]) * dinv + bg_ref[...], 0.0)
    out_ref[...] = (
        lax.dot_general(h, wl_ref[...], (((0,), (0,)), ((), ())),
                        preferred_element_type=jnp.float32)
        + bl_ref[...]
    )


def kernel(x, edge_index, W_g, b_g, W_l, b_l):
    src = edge_index[0].astype(jnp.int32)
    dst = edge_index[1].astype(jnp.int32)
    x_p = jnp.concatenate(
        [x, jnp.zeros((NROW - N_NODES, D_IN), jnp.float32)], axis=0
    )
    n_edges = src.shape[0]
    nch = -(-n_edges // EC)
    nch = -(-nch // NW) * NW
    pad = nch * EC - n_edges

    garbage = N_NODES + (jnp.arange(pad, dtype=jnp.int32) % 16)
    src_p = jnp.concatenate([src, jnp.zeros((pad,), jnp.int32)])
    dst_p = jnp.concatenate([dst, garbage])
    sd_f = (lax.shift_left(src_p, 16) | dst_p).reshape(nch, 16, 128)

    degp = _sc_degree(sd_f, nch).reshape(NC, NS, 1, NROW)

    yT = pl.pallas_call(
        _tc_y_body,
        grid=(TC_GRID,),
        in_specs=[
            pl.BlockSpec((NC, NS, 1, TC_BLK), lambda i: (0, 0, 0, i)),
            pl.BlockSpec((TC_BLK, D_IN), lambda i: (i, 0)),
            pl.BlockSpec((D_IN, D_HID), lambda i: (0, 0)),
        ],
        out_specs=pl.BlockSpec((D_HID, TC_BLK), lambda i: (0, i)),
        out_shape=jax.ShapeDtypeStruct((D_HID, NROW), jnp.float32),
    )(degp, x_p, W_g)

    p = _sc_aggregate(yT.reshape(NW, COLS * NROW), sd_f, nch)
    aggT = p.reshape(D_HID, NROW)

    out = pl.pallas_call(
        _tc_head_body,
        grid=(TC_GRID,),
        in_specs=[
            pl.BlockSpec((NC, NS, 1, TC_BLK), lambda i: (0, 0, 0, i)),
            pl.BlockSpec((D_HID, TC_BLK), lambda i: (0, i)),
            pl.BlockSpec((D_HID, TC_BLK), lambda i: (0, i)),
            pl.BlockSpec((D_HID, 1), lambda i: (0, 0)),
            pl.BlockSpec((D_HID, D_OUT), lambda i: (0, 0)),
            pl.BlockSpec((1, D_OUT), lambda i: (0, 0)),
        ],
        out_specs=pl.BlockSpec((TC_BLK, D_OUT), lambda i: (i, 0)),
        out_shape=jax.ShapeDtypeStruct((NROW, D_OUT), jnp.float32),
    )(degp, aggT, yT, b_g.reshape(D_HID, 1), W_l, b_l.reshape(1, D_OUT))

    return out[:N_NODES]
